# Initial kernel scaffold; baseline (speedup 1.0000x reference)
import jax
import jax.numpy as jnp
from jax.experimental import pallas as pl


def kernel(x):
    raise NotImplementedError("write your pallas kernel here")


# speedup vs baseline: 2.2288x; 2.2288x over previous
import dataclasses
import functools

import jax
import jax.numpy as jnp
from jax import lax
from jax.experimental import pallas as pl
from jax.experimental.pallas import tpu as pltpu
from jax.experimental.pallas import tpu_sc as plsc

ROWS = 64
N = 32768
TOPK = 64
L = 16
NCHUNK = N // L
NB = 4096
NB3 = 256
NWORKERS = 32
ROWS_PER_W = ROWS // NWORKERS


def _monotone(xi):
    return xi ^ (jnp.int32(0x7FFFFFFF) & (xi >> 31))


def _remap_chunk(xv, i):
    xx = xv[pl.ds(i, L)]
    return xx, _monotone(plsc.bitcast(xx, jnp.int32))


def _scan_hist_desc(hist, nb, want):
    iota = lax.iota(jnp.int32, L)

    def cond(carry):
        c, acc, B, r, nB, done = carry
        return jnp.logical_not(done) & (c >= 0)

    def body(carry):
        c, acc, B, r, nB, done = carry
        h = hist[pl.ds(c * L, L)]
        tot = jnp.sum(h)
        hr = lax.rev(h, (0,))
        cr = jnp.cumsum(hr)
        hit = (cr + acc) >= want
        nset = jnp.sum(hit.astype(jnp.int32))
        idx = L - nset
        sel = iota == idx
        cr_at = jnp.sum(jnp.where(sel, cr, 0))
        h_at = jnp.sum(jnp.where(sel, hr, 0))
        found = (acc + tot) >= want
        B_new = c * L + (L - 1) - idx
        r_new = want - (acc + cr_at - h_at)
        B = jnp.where(found, B_new, B)
        r = jnp.where(found, r_new, r)
        nB = jnp.where(found, h_at, nB)
        return (c - 1, acc + tot, B, r, nB, done | found)

    z = jnp.int32(0)
    init = (jnp.int32(nb // L - 1), z, z, z, z, jnp.bool_(False))
    out = lax.while_loop(cond, body, init)
    return out[2], out[3], out[4]


def _zero_hist(hist, nb):
    zeros_i = jnp.zeros((L,), jnp.int32)

    @pl.loop(0, nb, step=L)
    def _(i):
        hist[pl.ds(i, L)] = zeros_i


def _do_row(x_hbm, o_hbm, xv, hist, row):
    pltpu.sync_copy(x_hbm.at[row], xv)
    ones = jnp.ones((L,), jnp.int32)

    _zero_hist(hist, NB)

    @pl.loop(0, N, step=L)
    def _(i):
        _, v = _remap_chunk(xv, i)
        b = (v >> 20) + 2048
        plsc.addupdate_scatter(hist, [b], ones)

    B1, r1, _n1 = _scan_hist_desc(hist, NB, jnp.int32(TOPK))

    _zero_hist(hist, NB)

    @pl.loop(0, N, step=L)
    def _(i):
        _, v = _remap_chunk(xv, i)
        m = ((v >> 20) + 2048) == B1
        b2 = (v >> 8) & 0xFFF
        plsc.addupdate_scatter(hist, [b2], ones, mask=m)

    B2, r2, _n2 = _scan_hist_desc(hist, NB, r1)

    _zero_hist(hist, NB3)
    prefix = ((B1 - 2048) << 20) + (B2 << 8)

    @pl.loop(0, N, step=L)
    def _(i):
        _, v = _remap_chunk(xv, i)
        m = (v >> 8) == (prefix >> 8)
        b3 = v & 0xFF
        plsc.addupdate_scatter(hist, [b3], ones, mask=m)

    B3, r3, neq = _scan_hist_desc(hist, NB3, r2)
    vt = prefix + B3
    surplus = neq - r3

    @pl.when(surplus > 0)
    def _():
        def body(c, left):
            cc = NCHUNK - 1 - c
            xx, v = _remap_chunk(xv, cc * L)
            er = lax.rev((v == vt).astype(jnp.int32), (0,))
            pc = jnp.cumsum(er)
            zr = er * (pc <= left).astype(jnp.int32)
            zmask = lax.rev(zr, (0,)) > 0
            xv[pl.ds(cc * L, L)] = jnp.where(zmask, jnp.float32(0.0), xx)
            return left - jnp.sum(zr)

        lax.fori_loop(0, NCHUNK, body, surplus)

    @pl.loop(0, N, step=L)
    def _(i):
        xx, v = _remap_chunk(xv, i)
        xv[pl.ds(i, L)] = jnp.where(v >= vt, xx, jnp.float32(0.0))

    pltpu.sync_copy(xv, o_hbm.at[row])


def kernel(x):
    mesh = plsc.VectorSubcoreMesh(core_axis_name="c", subcore_axis_name="s")
    cp = pltpu.CompilerParams()
    if "needs_layout_passes" in pltpu.CompilerParams.__dataclass_fields__:
        cp = dataclasses.replace(cp, needs_layout_passes=False)

    @functools.partial(
        pl.kernel,
        out_type=jax.ShapeDtypeStruct((ROWS, N), jnp.float32),
        mesh=mesh,
        compiler_params=cp,
        scratch_types=[
            pltpu.VMEM((N,), jnp.float32),
            pltpu.VMEM((NB,), jnp.int32),
        ],
    )
    def _topk_mask(x_hbm, o_hbm, xv, hist):
        wid = lax.axis_index("s") * 2 + lax.axis_index("c")
        for j in range(ROWS_PER_W):
            _do_row(x_hbm, o_hbm, xv, hist, wid * ROWS_PER_W + j)

    return _topk_mask(x)


# speedup vs baseline: 3.7527x; 1.6837x over previous
import dataclasses
import functools

import jax
import jax.numpy as jnp
from jax import lax
from jax.experimental import pallas as pl
from jax.experimental.pallas import tpu as pltpu
from jax.experimental.pallas import tpu_sc as plsc

ROWS = 64
N = 32768
TOPK = 64
L = 16
NCHUNK = N // L
NB = 4096
NC = NB // L
NB3 = 256
NWORKERS = 32
ROWS_PER_W = ROWS // NWORKERS
UNROLL = 8


def _monotone(xi):
    return xi ^ (jnp.int32(0x7FFFFFFF) & (xi >> 31))


def _remap_chunk(xv, i):
    xx = xv[pl.ds(i, L)]
    return xx, _monotone(plsc.bitcast(xx, jnp.int32))


def _scan_fine_chunk(hist, c, want):
    iota = lax.iota(jnp.int32, L)
    h = hist[pl.ds(c * L, L)]
    hr = lax.rev(h, (0,))
    cr = jnp.cumsum(hr)
    hit = cr >= want
    nset = jnp.sum(hit.astype(jnp.int32))
    idx = L - nset
    sel = iota == idx
    cr_at = jnp.sum(jnp.where(sel, cr, 0))
    h_at = jnp.sum(jnp.where(sel, hr, 0))
    B = c * L + (L - 1) - idx
    r = want - (cr_at - h_at)
    return B, r, h_at


def _scan_hist_desc(hist, nb, want):
    def cond(carry):
        c, acc, done = carry
        return jnp.logical_not(done) & (c >= 0)

    def body(carry):
        c, acc, done = carry
        tot = jnp.sum(hist[pl.ds(c * L, L)])
        found = (acc + tot) >= want
        return (c - 1, acc + jnp.where(found, 0, tot), done | found)

    c_end, acc, _ = lax.while_loop(
        cond, body, (jnp.int32(nb // L - 1), jnp.int32(0), jnp.bool_(False))
    )
    return _scan_fine_chunk(hist, c_end + 1, want - acc)


def _scan_two_level(hist, coarse, want):
    C, rC, _ = _scan_hist_desc(coarse, NC, want)
    return _scan_fine_chunk(hist, C, rC)


def _zero(ref, n):
    zeros_i = jnp.zeros((L,), jnp.int32)

    @pl.loop(0, n, step=L * 4)
    def _(i):
        for u in range(4):
            ref[pl.ds(i + u * L, L)] = zeros_i


def _exact_select(x_hbm, xv, hist, row, prefix, r2):
    ones = jnp.ones((L,), jnp.int32)
    pltpu.sync_copy(x_hbm.at[row], xv)
    _zero(hist, NB3)

    @pl.loop(0, N, step=L)
    def _(i):
        _, v = _remap_chunk(xv, i)
        m = (v >> 8) == (prefix >> 8)
        plsc.addupdate_scatter(hist, [v & 0xFF], ones, mask=m)

    B3, r3, neq = _scan_hist_desc(hist, NB3, r2)
    vt = prefix + B3
    surplus = neq - r3

    @pl.when(surplus > 0)
    def _():
        def body(c, left):
            cc = NCHUNK - 1 - c
            xx, v = _remap_chunk(xv, cc * L)
            er = lax.rev((v == vt).astype(jnp.int32), (0,))
            pc = jnp.cumsum(er)
            zr = er * (pc <= left).astype(jnp.int32)
            zmask = lax.rev(zr, (0,)) > 0
            xv[pl.ds(cc * L, L)] = jnp.where(zmask, jnp.float32(0.0), xx)
            return left - jnp.sum(zr)

        lax.fori_loop(0, NCHUNK, body, surplus)

    @pl.loop(0, N, step=L)
    def _(i):
        xx, v = _remap_chunk(xv, i)
        xv[pl.ds(i, L)] = jnp.where(v >= vt, xx, jnp.float32(0.0))


def _do_row(x_hbm, xv, hist, coarse, row):
    ones = jnp.ones((L,), jnp.int32)

    _zero(hist, NB)
    _zero(coarse, NC)

    @pl.loop(0, N, step=L * UNROLL)
    def _(i):
        for u in range(UNROLL):
            _, v = _remap_chunk(xv, i + u * L)
            b = (v >> 20) + 2048
            plsc.addupdate_scatter(hist, [b], ones)
            plsc.addupdate_scatter(coarse, [b >> 4], ones)

    B1, r1, _n1 = _scan_two_level(hist, coarse, jnp.int32(TOPK))

    _zero(hist, NB)
    _zero(coarse, NC)

    @pl.loop(0, N, step=L * UNROLL)
    def _(i):
        for u in range(UNROLL):
            _, v = _remap_chunk(xv, i + u * L)
            m = (v >> 20) == (B1 - 2048)
            b2 = (v >> 8) & 0xFFF
            plsc.addupdate_scatter(hist, [b2], ones, mask=m)
            plsc.addupdate_scatter(coarse, [b2 >> 4], ones, mask=m)

    B2, r2, n2 = _scan_two_level(hist, coarse, r1)
    prefix = ((B1 - 2048) << 20) + (B2 << 8)
    surplus24 = n2 - r2

    @pl.when(surplus24 == 0)
    def _():
        @pl.loop(0, N, step=L * UNROLL)
        def _(i):
            for u in range(UNROLL):
                xx, v = _remap_chunk(xv, i + u * L)
                xv[pl.ds(i + u * L, L)] = jnp.where(
                    v >= prefix, xx, jnp.float32(0.0)
                )

    @pl.when(surplus24 > 0)
    def _():
        _exact_select(x_hbm, xv, hist, row, prefix, r2)


def kernel(x):
    mesh = plsc.VectorSubcoreMesh(core_axis_name="c", subcore_axis_name="s")
    cp = pltpu.CompilerParams()
    if "needs_layout_passes" in pltpu.CompilerParams.__dataclass_fields__:
        cp = dataclasses.replace(cp, needs_layout_passes=False)

    @functools.partial(
        pl.kernel,
        out_type=jax.ShapeDtypeStruct((ROWS, N), jnp.float32),
        mesh=mesh,
        compiler_params=cp,
        scratch_types=[
            pltpu.VMEM((N,), jnp.float32),
            pltpu.VMEM((N,), jnp.float32),
            pltpu.VMEM((NB,), jnp.int32),
            pltpu.VMEM((NC,), jnp.int32),
            pltpu.SemaphoreType.DMA,
            pltpu.SemaphoreType.DMA,
            pltpu.SemaphoreType.DMA,
            pltpu.SemaphoreType.DMA,
        ],
    )
    def _topk_mask(x_hbm, o_hbm, buf0, buf1, hist, coarse,
                   sin0, sin1, sout0, sout1):
        wid = lax.axis_index("s") * 2 + lax.axis_index("c")
        r0 = wid * ROWS_PER_W
        r1_ = r0 + 1
        in0 = pltpu.make_async_copy(x_hbm.at[r0], buf0, sin0)
        in1 = pltpu.make_async_copy(x_hbm.at[r1_], buf1, sin1)
        in0.start()
        in1.start()
        in0.wait()
        _do_row(x_hbm, buf0, hist, coarse, r0)
        out0 = pltpu.make_async_copy(buf0, o_hbm.at[r0], sout0)
        out0.start()
        in1.wait()
        _do_row(x_hbm, buf1, hist, coarse, r1_)
        out1 = pltpu.make_async_copy(buf1, o_hbm.at[r1_], sout1)
        out1.start()
        out0.wait()
        out1.wait()

    return _topk_mask(x)


# speedup vs baseline: 9.3634x; 2.4951x over previous
import dataclasses
import functools

import jax
import jax.numpy as jnp
from jax import lax
from jax.experimental import pallas as pl
from jax.experimental.pallas import tpu as pltpu
from jax.experimental.pallas import tpu_sc as plsc

ROWS = 64
N = 32768
TOPK = 64
L = 16
NCHUNK = N // L
NB = 4096
NC = NB // L
NB3 = 256
NWORKERS = 32
ROWS_PER_W = ROWS // NWORKERS
UNROLL = 8
INT_MIN = -(2**31)


def _monotone(xi):
    return xi ^ (jnp.int32(0x7FFFFFFF) & (xi >> 31))


def _remap_chunk(xv, i):
    xx = xv[pl.ds(i, L)]
    return xx, _monotone(plsc.bitcast(xx, jnp.int32))


def _scan_fine_chunk(hist, c, want):
    iota = lax.iota(jnp.int32, L)
    h = hist[pl.ds(c * L, L)]
    hr = lax.rev(h, (0,))
    cr = jnp.cumsum(hr)
    hit = cr >= want
    nset = jnp.sum(hit.astype(jnp.int32))
    idx = L - nset
    sel = iota == idx
    cr_at = jnp.sum(jnp.where(sel, cr, 0))
    h_at = jnp.sum(jnp.where(sel, hr, 0))
    B = c * L + (L - 1) - idx
    r = want - (cr_at - h_at)
    return B, r, h_at


def _scan_hist_desc(hist, cstart, want):
    def cond(carry):
        c, acc, done = carry
        return jnp.logical_not(done) & (c >= 0)

    def body(carry):
        c, acc, done = carry
        tot = jnp.sum(hist[pl.ds(c * L, L)])
        found = (acc + tot) >= want
        return (c - 1, acc + jnp.where(found, 0, tot), done | found)

    c_end, acc, _ = lax.while_loop(
        cond, body, (cstart, jnp.int32(0), jnp.bool_(False))
    )
    return _scan_fine_chunk(hist, c_end + 1, want - acc)


def _zero(ref, n):
    zeros_i = jnp.zeros((L,), jnp.int32)

    @plsc.parallel_loop(0, n, step=L * 4)
    def _(i):
        for u in range(4):
            ref[pl.ds(i + u * L, L)] = zeros_i


def _exact_select(x_hbm, xv, hist, row, prefix, r2):
    ones = jnp.ones((L,), jnp.int32)
    pltpu.sync_copy(x_hbm.at[row], xv)
    _zero(hist, NB3)

    @pl.loop(0, N, step=L)
    def _(i):
        _, v = _remap_chunk(xv, i)
        m = (v >> 8) == (prefix >> 8)
        plsc.addupdate_scatter(hist, [v & 0xFF], ones, mask=m)

    B3, r3, neq = _scan_hist_desc(hist, jnp.int32(NB3 // L - 1), r2)
    vt = prefix + B3
    surplus = neq - r3

    @pl.when(surplus > 0)
    def _():
        def body(c, left):
            cc = NCHUNK - 1 - c
            xx, v = _remap_chunk(xv, cc * L)
            er = lax.rev((v == vt).astype(jnp.int32), (0,))
            pc = jnp.cumsum(er)
            zr = er * (pc <= left).astype(jnp.int32)
            zmask = lax.rev(zr, (0,)) > 0
            xv[pl.ds(cc * L, L)] = jnp.where(zmask, jnp.float32(0.0), xx)
            return left - jnp.sum(zr)

        lax.fori_loop(0, NCHUNK, body, surplus)

    @pl.loop(0, N, step=L)
    def _(i):
        xx, v = _remap_chunk(xv, i)
        xv[pl.ds(i, L)] = jnp.where(v >= vt, xx, jnp.float32(0.0))


def _do_row(x_hbm, xv, hist, coarse, row):
    ones = jnp.ones((L,), jnp.int32)

    _zero(hist, NB)

    @plsc.parallel_loop(0, N, step=L * UNROLL,
                        carry=jnp.full((L,), INT_MIN, jnp.int32))
    def _p1(i, vmax):
        for u in range(UNROLL):
            _, v = _remap_chunk(xv, i + u * L)
            plsc.addupdate_scatter(hist, [(v >> 20) + 2048], ones)
            vmax = jnp.maximum(vmax, v)
        return vmax

    bmax = (jnp.max(_p1) >> 20) + 2048
    B1, r1, _n1 = _scan_hist_desc(hist, bmax >> 4, jnp.int32(TOPK))

    _zero(hist, NB)
    _zero(coarse, NC)

    @plsc.parallel_loop(0, N, step=L * UNROLL)
    def _(i):
        for u in range(UNROLL):
            _, v = _remap_chunk(xv, i + u * L)
            m = (v >> 20) == (B1 - 2048)
            b2 = (v >> 8) & 0xFFF
            plsc.addupdate_scatter(hist, [b2], ones, mask=m)
            plsc.addupdate_scatter(coarse, [b2 >> 4], ones, mask=m)

    C2, rC2, _ = _scan_hist_desc(coarse, jnp.int32(NC // L - 1), r1)
    B2, r2, n2 = _scan_fine_chunk(hist, C2, rC2)
    prefix = ((B1 - 2048) << 20) + (B2 << 8)
    surplus24 = n2 - r2

    @pl.when(surplus24 == 0)
    def _():
        @plsc.parallel_loop(0, N, step=L * UNROLL)
        def _(i):
            for u in range(UNROLL):
                xx, v = _remap_chunk(xv, i + u * L)
                xv[pl.ds(i + u * L, L)] = jnp.where(
                    v >= prefix, xx, jnp.float32(0.0)
                )

    @pl.when(surplus24 > 0)
    def _():
        _exact_select(x_hbm, xv, hist, row, prefix, r2)


def kernel(x):
    mesh = plsc.VectorSubcoreMesh(core_axis_name="c", subcore_axis_name="s")
    cp = pltpu.CompilerParams()
    if "needs_layout_passes" in pltpu.CompilerParams.__dataclass_fields__:
        cp = dataclasses.replace(cp, needs_layout_passes=False)

    @functools.partial(
        pl.kernel,
        out_type=jax.ShapeDtypeStruct((ROWS, N), jnp.float32),
        mesh=mesh,
        compiler_params=cp,
        scratch_types=[
            pltpu.VMEM((N,), jnp.float32),
            pltpu.VMEM((N,), jnp.float32),
            pltpu.VMEM((NB,), jnp.int32),
            pltpu.VMEM((NC,), jnp.int32),
            pltpu.SemaphoreType.DMA,
            pltpu.SemaphoreType.DMA,
            pltpu.SemaphoreType.DMA,
            pltpu.SemaphoreType.DMA,
        ],
    )
    def _topk_mask(x_hbm, o_hbm, buf0, buf1, hist, coarse,
                   sin0, sin1, sout0, sout1):
        wid = lax.axis_index("s") * 2 + lax.axis_index("c")
        r0 = wid * ROWS_PER_W
        r1_ = r0 + 1
        in0 = pltpu.make_async_copy(x_hbm.at[r0], buf0, sin0)
        in1 = pltpu.make_async_copy(x_hbm.at[r1_], buf1, sin1)
        in0.start()
        in1.start()
        in0.wait()
        _do_row(x_hbm, buf0, hist, coarse, r0)
        out0 = pltpu.make_async_copy(buf0, o_hbm.at[r0], sout0)
        out0.start()
        in1.wait()
        _do_row(x_hbm, buf1, hist, coarse, r1_)
        out1 = pltpu.make_async_copy(buf1, o_hbm.at[r1_], sout1)
        out1.start()
        out0.wait()
        out1.wait()

    return _topk_mask(x)
